# baseline (speedup 1.0000x reference)
import functools

import jax
import jax.numpy as jnp
from jax import lax
from jax.experimental import pallas as pl
from jax.experimental.pallas import tpu as pltpu
from jax.experimental.pallas import tpu_sc as plsc

_R = 128


@functools.lru_cache(maxsize=None)
def _make_retriever(L, D):
    NC, NS, LANES = 2, 16, 16
    NW = NC * NS
    T = 2 * _R + 1
    q_per_tile = L // NW
    SEG = 16
    K_SEG = L // SEG
    A = D // 8
    P_BUILD = K_SEG + q_per_tile
    STRIDE = 33
    NBUF = 16
    ROWS_PER_IT = NBUF // A

    mesh = plsc.VectorSubcoreMesh(core_axis_name="c", subcore_axis_name="s")

    @functools.partial(
        pl.kernel,
        out_type=jax.ShapeDtypeStruct((L, L * D // 1024, 8, 128), jnp.float32),
        mesh=mesh,
        compiler_params=pltpu.CompilerParams(needs_layout_passes=False),
        scratch_types=[
            pltpu.VMEM((T, D), jnp.float32),
            pltpu.VMEM((P_BUILD * STRIDE,), jnp.float32),
            pltpu.VMEM((NBUF, 8, 128), jnp.float32),
            pltpu.SemaphoreType.DMA,
        ],
    )
    def retrieve(table_hbm, out_hbm, tbl_v, band, slabs, sem_w):
        cid = lax.axis_index("c")
        sid = lax.axis_index("s")
        wid = sid * NC + cid
        q0 = wid * q_per_tile
        g0 = (L - q_per_tile) - q0

        pltpu.sync_copy(table_hbm, tbl_v)

        lane = lax.broadcasted_iota(jnp.int32, (LANES,), 0)
        lane_s = lane * STRIDE

        def seg_body(seg, carry):
            gbase = g0 + seg * K_SEG

            def build_g(g, bc):
                tidx = jnp.clip(gbase + g - (L - 1 - _R), 0, 2 * _R)
                base = g * STRIDE
                band[pl.ds(g * STRIDE, LANES)] = tbl_v[tidx, pl.ds(0, LANES)]
                band[pl.ds(g * STRIDE + LANES, LANES)] = tbl_v[tidx, pl.ds(LANES, LANES)]
                return bc

            lax.fori_loop(0, P_BUILD, build_g, 0)

            def it_body(it, bc):
                descs = []
                for rr in range(ROWS_PER_IT):
                    r = it * ROWS_PER_IT + rr
                    o = (q_per_tile - 1) - r
                    for a in range(A):
                        buf = rr * A + a
                        for c2 in range(0, 8, 4):
                            vals = []
                            for c in (c2, c2 + 1, c2 + 2, c2 + 3):
                                base = o * STRIDE + a * 8 + c
                                for e0 in range(0, 128, LANES):
                                    vals.append((c, e0, plsc.load_gather(
                                        band, [lane_s + (base + e0 * STRIDE)]
                                    )))
                            for c, e0, v in vals:
                                slabs[buf, c, pl.ds(e0, LANES)] = v
                        descs.append(
                            pltpu.async_copy(
                                slabs.at[buf],
                                out_hbm.at[q0 + r, a * SEG + seg],
                                sem_w,
                            )
                        )
                for dsc in descs:
                    dsc.wait()
                return bc

            lax.fori_loop(0, q_per_tile // ROWS_PER_IT, it_body, 0)
            return carry

        lax.fori_loop(0, SEG, seg_body, 0)

    return retrieve


def kernel(x, rel_pos_embeddings):
    L = x.shape[1]
    D = rel_pos_embeddings.shape[1]
    t6 = _make_retriever(L, D)(rel_pos_embeddings)
    t5 = t6.reshape(L, D // 8, L // 128, 8, 128)
    return jnp.transpose(t5, (0, 2, 4, 1, 3)).reshape(L, L, D)


# speedup vs baseline: 3.3245x; 3.3245x over previous
import functools

import jax
import jax.numpy as jnp
from jax import lax
from jax.experimental import pallas as pl
from jax.experimental.pallas import tpu as pltpu
from jax.experimental.pallas import tpu_sc as plsc

_R = 128


@functools.lru_cache(maxsize=None)
def _make_retriever(L, D):
    NC, NS, LANES = 2, 16, 16
    NW = NC * NS
    T = 2 * _R + 1
    q_per_tile = L // NW
    SEG = 16
    K_SEG = L // SEG
    A = D // 8
    P_BUILD = K_SEG + q_per_tile
    STRIDE = 33
    NBUF = 8
    ROWS_PER_IT = NBUF // A
    LO_MAX = (L - 1 - _R) - (K_SEG - 1)
    HI_MIN = L - 1 + _R

    mesh = plsc.VectorSubcoreMesh(core_axis_name="c", subcore_axis_name="s")

    @functools.partial(
        pl.kernel,
        out_type=jax.ShapeDtypeStruct((L, L * D // 1024, 8, 128), jnp.float32),
        mesh=mesh,
        compiler_params=pltpu.CompilerParams(needs_layout_passes=False),
        scratch_types=[
            pltpu.VMEM((T, D), jnp.float32),
            pltpu.VMEM((P_BUILD * STRIDE,), jnp.float32),
            pltpu.VMEM((NBUF, 8, 128), jnp.float32),
            pltpu.VMEM((2 * A, 8, 128), jnp.float32),
            pltpu.SemaphoreType.DMA,
        ],
    )
    def retrieve(table_hbm, out_hbm, tbl_v, band, slabs, cslabs, sem_w):
        cid = lax.axis_index("c")
        sid = lax.axis_index("s")
        wid = sid * NC + cid
        q0 = wid * q_per_tile
        g0 = (L - q_per_tile) - q0

        pltpu.sync_copy(table_hbm, tbl_v)

        lane = lax.broadcasted_iota(jnp.int32, (LANES,), 0)
        lane_s = lane * STRIDE

        row_lo = [tbl_v[0, pl.ds(h, LANES)] for h in range(0, D, LANES)]
        row_hi = [tbl_v[T - 1, pl.ds(h, LANES)] for h in range(0, D, LANES)]
        for a in range(A):
            for c in range(8):
                d = a * 8 + c
                vlo = jnp.full((LANES,), row_lo[d // LANES][d % LANES], jnp.float32)
                vhi = jnp.full((LANES,), row_hi[d // LANES][d % LANES], jnp.float32)
                for e0 in range(0, K_SEG, LANES):
                    cslabs[a, c, pl.ds(e0, LANES)] = vlo
                    cslabs[A + a, c, pl.ds(e0, LANES)] = vhi

        def drain_one(b):
            pltpu.make_async_copy(
                out_hbm.at[q0].at[0], slabs.at[b], sem_w
            ).wait()

        def seg_body(seg, carry):
            gbase = g0 + seg * K_SEG

            def build_g(g, bc):
                tidx = jnp.clip(gbase + g - (L - 1 - _R), 0, 2 * _R)
                band[pl.ds(g * STRIDE, LANES)] = tbl_v[tidx, pl.ds(0, LANES)]
                band[pl.ds(g * STRIDE + LANES, LANES)] = tbl_v[
                    tidx, pl.ds(LANES, LANES)
                ]
                return bc

            lax.fori_loop(0, P_BUILD, build_g, 0)

            def it_body(it, bc):
                @pl.when(seg * (q_per_tile // ROWS_PER_IT) + it > 0)
                def _drain_prev():
                    for b in range(NBUF):
                        drain_one(b)

                for rr in range(ROWS_PER_IT):
                    r = it * ROWS_PER_IT + rr
                    o = (q_per_tile - 1) - r
                    S = gbase + o
                    for a in range(A):
                        buf = rr * A + a
                        dst = out_hbm.at[q0 + r, a * SEG + seg]

                        @pl.when(S <= LO_MAX)
                        def _const_lo(a=a, dst=dst):
                            pltpu.async_copy(cslabs.at[a], dst, sem_w)

                        @pl.when(S >= HI_MIN)
                        def _const_hi(a=a, dst=dst):
                            pltpu.async_copy(cslabs.at[A + a], dst, sem_w)

                        @pl.when(jnp.logical_and(S > LO_MAX, S < HI_MIN))
                        def _mixed(a=a, buf=buf, dst=dst, o=o):
                            for c2 in range(0, 8, 2):
                                vals = []
                                for c in (c2, c2 + 1):
                                    base = o * STRIDE + a * 8 + c
                                    for e0 in range(0, K_SEG, LANES):
                                        vals.append((c, e0, plsc.load_gather(
                                            band, [lane_s + (base + e0 * STRIDE)]
                                        )))
                                for c, e0, v in vals:
                                    slabs[buf, c, pl.ds(e0, LANES)] = v
                            pltpu.async_copy(slabs.at[buf], dst, sem_w)
                return bc

            lax.fori_loop(0, q_per_tile // ROWS_PER_IT, it_body, 0)
            return carry

        lax.fori_loop(0, SEG, seg_body, 0)

        for b in range(NBUF):
            drain_one(b)

    return retrieve


def kernel(x, rel_pos_embeddings):
    L = x.shape[1]
    D = rel_pos_embeddings.shape[1]
    t6 = _make_retriever(L, D)(rel_pos_embeddings)
    t5 = t6.reshape(L, D // 8, L // 128, 8, 128)
    return jnp.transpose(t5, (0, 2, 4, 1, 3)).reshape(L, L, D)


# speedup vs baseline: 3.4936x; 1.0509x over previous
import functools

import jax
import jax.numpy as jnp
from jax import lax
from jax.experimental import pallas as pl
from jax.experimental.pallas import tpu as pltpu
from jax.experimental.pallas import tpu_sc as plsc

_R = 128


@functools.lru_cache(maxsize=None)
def _make_retriever(L, D):
    NC, NS, LANES = 2, 16, 16
    NW = NC * NS
    T = 2 * _R + 1
    q_per_tile = L // NW
    SEG = 16
    K_SEG = L // SEG
    A = D // 8
    P_BUILD = K_SEG + q_per_tile
    STRIDE = 33
    NBUF = 8
    ROWS_PER_IT = NBUF // A
    LO_MAX = (L - 1 - _R) - (K_SEG - 1)
    HI_MIN = L - 1 + _R

    mesh = plsc.VectorSubcoreMesh(core_axis_name="c", subcore_axis_name="s")

    @functools.partial(
        pl.kernel,
        out_type=jax.ShapeDtypeStruct((L, L * D // 1024, 8, 128), jnp.float32),
        mesh=mesh,
        compiler_params=pltpu.CompilerParams(needs_layout_passes=False),
        scratch_types=[
            pltpu.VMEM((T, D), jnp.float32),
            pltpu.VMEM((P_BUILD * STRIDE,), jnp.float32),
            pltpu.VMEM((NBUF, 8, 128), jnp.float32),
            pltpu.VMEM((2 * A, 8, 128), jnp.float32),
            pltpu.SemaphoreType.DMA,
        ],
    )
    def retrieve(table_hbm, out_hbm, tbl_v, band, slabs, cslabs, sem_w):
        cid = lax.axis_index("c")
        sid = lax.axis_index("s")
        wid = sid * NC + cid
        q0 = wid * q_per_tile
        g0 = (L - q_per_tile) - q0

        pltpu.sync_copy(table_hbm, tbl_v)

        lane = lax.broadcasted_iota(jnp.int32, (LANES,), 0)
        lane_s = lane * STRIDE

        row_lo = [tbl_v[0, pl.ds(h, LANES)] for h in range(0, D, LANES)]
        row_hi = [tbl_v[T - 1, pl.ds(h, LANES)] for h in range(0, D, LANES)]
        for a in range(A):
            for c in range(8):
                d = a * 8 + c
                vlo = jnp.full((LANES,), row_lo[d // LANES][d % LANES], jnp.float32)
                vhi = jnp.full((LANES,), row_hi[d // LANES][d % LANES], jnp.float32)
                for e0 in range(0, K_SEG, LANES):
                    cslabs[a, c, pl.ds(e0, LANES)] = vlo
                    cslabs[A + a, c, pl.ds(e0, LANES)] = vhi

        def drain_one(b):
            pltpu.make_async_copy(
                out_hbm.at[q0].at[0], slabs.at[b], sem_w
            ).wait()

        def seg_body(seg, carry):
            gbase = g0 + seg * K_SEG

            def build_g(g, bc):
                tidx = jnp.clip(gbase + g - (L - 1 - _R), 0, 2 * _R)
                band[pl.ds(g * STRIDE, LANES)] = tbl_v[tidx, pl.ds(0, LANES)]
                band[pl.ds(g * STRIDE + LANES, LANES)] = tbl_v[
                    tidx, pl.ds(LANES, LANES)
                ]
                return bc

            @pl.when(jnp.logical_and(gbase + (q_per_tile - 1) > LO_MAX,
                                     gbase < HI_MIN))
            def _build():
                lax.fori_loop(0, P_BUILD, build_g, 0)

            def it_body(it, bc):
                @pl.when(seg * (q_per_tile // ROWS_PER_IT) + it > 0)
                def _drain_prev():
                    for b in range(NBUF):
                        drain_one(b)

                for rr in range(ROWS_PER_IT):
                    r = it * ROWS_PER_IT + rr
                    o = (q_per_tile - 1) - r
                    S = gbase + o
                    for a in range(A):
                        buf = rr * A + a
                        dst = out_hbm.at[q0 + r, a * SEG + seg]

                        @pl.when(S <= LO_MAX)
                        def _const_lo(a=a, dst=dst):
                            pltpu.async_copy(cslabs.at[a], dst, sem_w)

                        @pl.when(S >= HI_MIN)
                        def _const_hi(a=a, dst=dst):
                            pltpu.async_copy(cslabs.at[A + a], dst, sem_w)

                        @pl.when(jnp.logical_and(S > LO_MAX, S < HI_MIN))
                        def _mixed(a=a, buf=buf, dst=dst, o=o):
                            for c2 in range(0, 8, 2):
                                vals = []
                                for c in (c2, c2 + 1):
                                    base = o * STRIDE + a * 8 + c
                                    for e0 in range(0, K_SEG, LANES):
                                        vals.append((c, e0, plsc.load_gather(
                                            band, [lane_s + (base + e0 * STRIDE)]
                                        )))
                                for c, e0, v in vals:
                                    slabs[buf, c, pl.ds(e0, LANES)] = v
                            pltpu.async_copy(slabs.at[buf], dst, sem_w)
                return bc

            lax.fori_loop(0, q_per_tile // ROWS_PER_IT, it_body, 0)
            return carry

        lax.fori_loop(0, SEG, seg_body, 0)

        for b in range(NBUF):
            drain_one(b)

    return retrieve


def kernel(x, rel_pos_embeddings):
    L = x.shape[1]
    D = rel_pos_embeddings.shape[1]
    t6 = _make_retriever(L, D)(rel_pos_embeddings)
    t5 = t6.reshape(L, D // 8, L // 128, 8, 128)
    return jnp.transpose(t5, (0, 2, 4, 1, 3)).reshape(L, L, D)


# speedup vs baseline: 3.4972x; 1.0010x over previous
import functools

import jax
import jax.numpy as jnp
from jax import lax
from jax.experimental import pallas as pl
from jax.experimental.pallas import tpu as pltpu
from jax.experimental.pallas import tpu_sc as plsc

_R = 128


@functools.lru_cache(maxsize=None)
def _make_retriever(L, D):
    NC, NS, LANES = 2, 16, 16
    NW = NC * NS
    T = 2 * _R + 1
    q_per_tile = L // NW
    SEG = 16
    K_SEG = L // SEG
    A = D // 8
    P_BUILD = K_SEG + q_per_tile
    STRIDE = 33
    NBUF = 8
    ROWS_PER_IT = NBUF // A
    LO_MAX = (L - 1 - _R) - (K_SEG - 1)
    HI_MIN = L - 1 + _R

    mesh = plsc.VectorSubcoreMesh(core_axis_name="c", subcore_axis_name="s")

    @functools.partial(
        pl.kernel,
        out_type=jax.ShapeDtypeStruct((L, L * D // 1024, 8, 128), jnp.float32),
        mesh=mesh,
        compiler_params=pltpu.CompilerParams(needs_layout_passes=False),
        scratch_types=[
            pltpu.VMEM((T, D), jnp.float32),
            pltpu.VMEM((P_BUILD * STRIDE,), jnp.float32),
            pltpu.VMEM((NBUF, 8, 128), jnp.float32),
            pltpu.VMEM((2 * A, 8, 128), jnp.float32),
            pltpu.SemaphoreType.DMA,
        ],
    )
    def retrieve(table_hbm, out_hbm, tbl_v, band, slabs, cslabs, sem_w):
        cid = lax.axis_index("c")
        sid = lax.axis_index("s")
        wid = sid * NC + cid
        q0 = wid * q_per_tile
        g0 = (L - q_per_tile) - q0

        pltpu.sync_copy(table_hbm, tbl_v)

        lane = lax.broadcasted_iota(jnp.int32, (LANES,), 0)
        lane_s = lane * STRIDE

        row_lo = [tbl_v[0, pl.ds(h, LANES)] for h in range(0, D, LANES)]
        row_hi = [tbl_v[T - 1, pl.ds(h, LANES)] for h in range(0, D, LANES)]
        for a in range(A):
            for c in range(8):
                d = a * 8 + c
                vlo = jnp.full((LANES,), row_lo[d // LANES][d % LANES], jnp.float32)
                vhi = jnp.full((LANES,), row_hi[d // LANES][d % LANES], jnp.float32)
                for e0 in range(0, K_SEG, LANES):
                    cslabs[a, c, pl.ds(e0, LANES)] = vlo
                    cslabs[A + a, c, pl.ds(e0, LANES)] = vhi

        def drain_nbuf():
            pltpu.make_async_copy(
                out_hbm.at[q0].at[pl.ds(0, NBUF)], slabs, sem_w
            ).wait()

        def seg_body(seg, carry):
            gbase = g0 + seg * K_SEG

            def build_g(g, bc):
                tidx = jnp.clip(gbase + g - (L - 1 - _R), 0, 2 * _R)
                band[pl.ds(g * STRIDE, LANES)] = tbl_v[tidx, pl.ds(0, LANES)]
                band[pl.ds(g * STRIDE + LANES, LANES)] = tbl_v[
                    tidx, pl.ds(LANES, LANES)
                ]
                return bc

            @pl.when(jnp.logical_and(gbase + (q_per_tile - 1) > LO_MAX,
                                     gbase < HI_MIN))
            def _build():
                lax.fori_loop(0, P_BUILD, build_g, 0)

            def it_body(it, bc):
                @pl.when(seg * (q_per_tile // ROWS_PER_IT) + it > 0)
                def _drain_prev():
                    drain_nbuf()

                for rr in range(ROWS_PER_IT):
                    r = it * ROWS_PER_IT + rr
                    o = (q_per_tile - 1) - r
                    S = gbase + o
                    for a in range(A):
                        buf = rr * A + a
                        dst = out_hbm.at[q0 + r, a * SEG + seg]

                        @pl.when(S <= LO_MAX)
                        def _const_lo(a=a, dst=dst):
                            pltpu.async_copy(cslabs.at[a], dst, sem_w)

                        @pl.when(S >= HI_MIN)
                        def _const_hi(a=a, dst=dst):
                            pltpu.async_copy(cslabs.at[A + a], dst, sem_w)

                        @pl.when(jnp.logical_and(S > LO_MAX, S < HI_MIN))
                        def _mixed(a=a, buf=buf, dst=dst, o=o):
                            for c2 in range(0, 8, 2):
                                vals = []
                                for c in (c2, c2 + 1):
                                    base = o * STRIDE + a * 8 + c
                                    for e0 in range(0, K_SEG, LANES):
                                        vals.append((c, e0, plsc.load_gather(
                                            band, [lane_s + (base + e0 * STRIDE)]
                                        )))
                                for c, e0, v in vals:
                                    slabs[buf, c, pl.ds(e0, LANES)] = v
                            pltpu.async_copy(slabs.at[buf], dst, sem_w)
                return bc

            lax.fori_loop(0, q_per_tile // ROWS_PER_IT, it_body, 0)
            return carry

        lax.fori_loop(0, SEG, seg_body, 0)

        drain_nbuf()

    return retrieve


def kernel(x, rel_pos_embeddings):
    L = x.shape[1]
    D = rel_pos_embeddings.shape[1]
    t6 = _make_retriever(L, D)(rel_pos_embeddings)
    t5 = t6.reshape(L, D // 8, L // 128, 8, 128)
    return jnp.transpose(t5, (0, 2, 4, 1, 3)).reshape(L, L, D)
